# baseline (speedup 1.0000x reference)
import functools

import jax
import jax.numpy as jnp
from jax import lax
from jax.experimental import pallas as pl
from jax.experimental.pallas import tpu as pltpu
from jax.experimental.pallas import tpu_sc as plsc

HIDDEN = 128
CHUNK = 128
LANES = 16
NBUF = 3
NWORKERS = 32


def _sc_kernel_body(nf_hbm, pos_hbm, pe_hbm, out_hbm,
                    pos_v, idx_v, rows_v, nf_v,
                    sem_pos, sem_g, sem_st):
    sid = lax.axis_index("s")
    wid = sid * 2 + lax.axis_index("c")
    n_chunks = nf_hbm.shape[0] // CHUNK
    kw = n_chunks // NWORKERS + jnp.where(wid < n_chunks % NWORKERS, 1, 0)


    def chunk_of(k):
        return k * NWORKERS + wid

    def s0(k):
        @pl.when(k < kw)
        def _():
            b = lax.rem(k, NBUF)
            pltpu.async_copy(pos_hbm.at[pl.ds(chunk_of(k) * CHUNK, CHUNK)],
                             pos_v.at[b], sem_pos.at[b])

    def s1(k):
        @pl.when(k < kw)
        def _():
            b = lax.rem(k, NBUF)
            pltpu.make_async_copy(pos_hbm.at[pl.ds(0, CHUNK)],
                                  pos_v.at[b], sem_pos.at[b]).wait()
            for i in range(CHUNK // LANES):
                p = pos_v[b, pl.ds(i * LANES, LANES)]
                idx_v[b, pl.ds(i * LANES, LANES)] = (
                    jnp.clip((p * 999.0).astype(jnp.int32), 0, 999))

            @pl.when(k >= NBUF)
            def _():
                pltpu.make_async_copy(rows_v.at[b],
                                      out_hbm.at[pl.ds(0, CHUNK)],
                                      sem_st.at[b]).wait()

            pltpu.async_copy(pe_hbm.at[idx_v.at[b]], rows_v.at[b], sem_g.at[b])
            pltpu.async_copy(nf_hbm.at[pl.ds(chunk_of(k) * CHUNK, CHUNK)],
                             nf_v.at[b], sem_g.at[b])

    def s2(k):
        @pl.when(k < kw)
        def _():
            b = lax.rem(k, NBUF)
            pltpu.make_async_copy(pe_hbm.at[pl.ds(0, CHUNK)],
                                  rows_v.at[b], sem_g.at[b]).wait()
            pltpu.make_async_copy(nf_hbm.at[pl.ds(0, CHUNK)],
                                  nf_v.at[b], sem_g.at[b]).wait()

            @plsc.parallel_loop(0, CHUNK, step=1, unroll=4)
            def _(r):
                for j in range(HIDDEN // LANES):
                    plsc.addupdate(rows_v.at[b, r, pl.ds(j * LANES, LANES)],
                                   nf_v[b, r, pl.ds(j * LANES, LANES)])

            pltpu.async_copy(rows_v.at[b],
                             out_hbm.at[pl.ds(chunk_of(k) * CHUNK, CHUNK)],
                             sem_st.at[b])

    s0(jnp.int32(0))
    s0(jnp.int32(1))
    s1(jnp.int32(0))

    def main_body(k, carry):
        s0(k + 2)
        s1(k + 1)
        s2(k)
        return carry

    lax.fori_loop(0, kw, main_body, 0)

    for db in range(NBUF):
        b = lax.rem(kw - NBUF + db, NBUF)
        pltpu.make_async_copy(rows_v.at[b], out_hbm.at[pl.ds(0, CHUNK)],
                              sem_st.at[b]).wait()


def _build_sc_call(n_rows):
    mesh = plsc.VectorSubcoreMesh(core_axis_name="c", subcore_axis_name="s")
    return pl.kernel(
        _sc_kernel_body,
        mesh=mesh,
        out_type=jax.ShapeDtypeStruct((n_rows, HIDDEN), jnp.float32),
        scratch_types=[
            pltpu.VMEM((NBUF, CHUNK), jnp.float32),
            pltpu.VMEM((NBUF, CHUNK), jnp.int32),
            pltpu.VMEM((NBUF, CHUNK, HIDDEN), jnp.float32),
            pltpu.VMEM((NBUF, CHUNK, HIDDEN), jnp.float32),
            pltpu.SemaphoreType.DMA((NBUF,)),
            pltpu.SemaphoreType.DMA((NBUF,)),
            pltpu.SemaphoreType.DMA((NBUF,)),
        ],
    )


def kernel(node_features, layer_positions, pe):
    b, n, h = node_features.shape
    nf = node_features.reshape(b * n, h)
    pos = layer_positions.reshape(b * n)
    table = pe[0]
    out = _build_sc_call(b * n)(nf, pos, table)
    return out.reshape(b, n, h)


# speedup vs baseline: 1.0168x; 1.0168x over previous
import functools

import jax
import jax.numpy as jnp
from jax import lax
from jax.experimental import pallas as pl
from jax.experimental.pallas import tpu as pltpu
from jax.experimental.pallas import tpu_sc as plsc

HIDDEN = 128
CHUNK = 128
LANES = 16
NBUF = 3
NWORKERS = 32
KMAX = 98
NBIG = 21


def _sc_kernel_body(nf_hbm, pos_hbm, pe_hbm, out_hbm,
                    pos_v, idx_v, rows_v, nf_v, sem_g, sem_nf, sem_st):
    sid = lax.axis_index("s")
    wid = sid * 2 + lax.axis_index("c")
    kw = jnp.where(wid < NBIG, KMAX, KMAX - 1)
    start_chunk = KMAX * wid - jnp.maximum(wid - NBIG, 0)
    base_row = start_chunk * CHUNK

    @pl.when(wid < NBIG)
    def _():
        pltpu.sync_copy(pos_hbm.at[pl.ds(base_row, KMAX * CHUNK)], pos_v)

    @pl.when(wid >= NBIG)
    def _():
        pltpu.sync_copy(pos_hbm.at[pl.ds(base_row, (KMAX - 1) * CHUNK)],
                        pos_v.at[pl.ds(0, (KMAX - 1) * CHUNK)])

    def idx_body(i, carry):
        p = pos_v[pl.ds(i * LANES, LANES)]
        idx_v[pl.ds(i * LANES, LANES)] = (
            jnp.clip((p * 999.0).astype(jnp.int32), 0, 999))
        return carry

    lax.fori_loop(0, KMAX * CHUNK // LANES, idx_body, 0)

    def s1(k):
        @pl.when(k < kw)
        def _():
            b = lax.rem(k, NBUF)

            @pl.when(k >= NBUF)
            def _():
                pltpu.make_async_copy(nf_v.at[b],
                                      out_hbm.at[pl.ds(0, CHUNK)],
                                      sem_st.at[b]).wait()

            pltpu.async_copy(pe_hbm.at[idx_v.at[pl.ds(k * CHUNK, CHUNK)]],
                             rows_v.at[b], sem_g.at[b])
            pltpu.async_copy(nf_hbm.at[pl.ds(base_row + k * CHUNK, CHUNK)],
                             nf_v.at[b], sem_nf.at[b])

    def s2(k):
        @pl.when(k < kw)
        def _():
            b = lax.rem(k, NBUF)
            pltpu.make_async_copy(pe_hbm.at[pl.ds(0, CHUNK)],
                                  rows_v.at[b], sem_g.at[b]).wait()
            pltpu.make_async_copy(nf_hbm.at[pl.ds(0, CHUNK)],
                                  nf_v.at[b], sem_nf.at[b]).wait()

            @plsc.parallel_loop(0, CHUNK, step=1, unroll=8)
            def _(r):
                for j in range(HIDDEN // LANES):
                    plsc.addupdate(nf_v.at[b, r, pl.ds(j * LANES, LANES)],
                                   rows_v[b, r, pl.ds(j * LANES, LANES)])

            pltpu.async_copy(nf_v.at[b],
                             out_hbm.at[pl.ds(base_row + k * CHUNK, CHUNK)],
                             sem_st.at[b])

    s1(jnp.int32(0))
    s1(jnp.int32(1))

    def main_body(k, carry):
        s1(k + 2)
        s2(k)
        return carry

    lax.fori_loop(0, kw, main_body, 0)

    for db in range(NBUF):
        b = lax.rem(kw - NBUF + db, NBUF)
        pltpu.make_async_copy(nf_v.at[b], out_hbm.at[pl.ds(0, CHUNK)],
                              sem_st.at[b]).wait()


def _build_sc_call(n_rows):
    mesh = plsc.VectorSubcoreMesh(core_axis_name="c", subcore_axis_name="s")
    return pl.kernel(
        _sc_kernel_body,
        mesh=mesh,
        out_type=jax.ShapeDtypeStruct((n_rows, HIDDEN), jnp.float32),
        scratch_types=[
            pltpu.VMEM((KMAX * CHUNK,), jnp.float32),
            pltpu.VMEM((KMAX * CHUNK,), jnp.int32),
            pltpu.VMEM((NBUF, CHUNK, HIDDEN), jnp.float32),
            pltpu.VMEM((NBUF, CHUNK, HIDDEN), jnp.float32),
            pltpu.SemaphoreType.DMA((NBUF,)),
            pltpu.SemaphoreType.DMA((NBUF,)),
            pltpu.SemaphoreType.DMA((NBUF,)),
        ],
    )


def kernel(node_features, layer_positions, pe):
    b, n, h = node_features.shape
    nf = node_features.reshape(b * n, h)
    pos = layer_positions.reshape(b * n)
    table = pe[0]
    out = _build_sc_call(b * n)(nf, pos, table)
    return out.reshape(b, n, h)
